# baseline (speedup 1.0000x reference)
import functools

import jax
import jax.numpy as jnp
from jax.experimental import pallas as pl
from jax.experimental.pallas import tpu as pltpu

HID = 128
NUM_LAYER = 2
BATCH = 4096


def _layer_update_body(h_ref, s_ref, cnt_ref, w_ref, o_ref):
    h = h_ref[...]
    s = s_ref[...]
    deg = jnp.maximum(cnt_ref[...][:, 0:1], 1.0)
    x = h + s / deg
    o_ref[...] = jnp.maximum(jnp.dot(x, w_ref[...], preferred_element_type=jnp.float32), 0.0)


def _layer_update(h, ssum, cnt, W, block=512):
    n = h.shape[0]
    grid = (pl.cdiv(n, block),)
    return pl.pallas_call(
        _layer_update_body,
        grid=grid,
        in_specs=[
            pl.BlockSpec((block, HID), lambda i: (i, 0)),
            pl.BlockSpec((block, HID), lambda i: (i, 0)),
            pl.BlockSpec((block, 16), lambda i: (i, 0)),
            pl.BlockSpec((HID, HID), lambda i: (0, 0)),
        ],
        out_specs=pl.BlockSpec((block, HID), lambda i: (i, 0)),
        out_shape=jax.ShapeDtypeStruct((n, HID), jnp.float32),
    )(h, ssum, cnt, W)


def _mlp_body(u1_ref, u2_ref, uc_ref, w1_ref, b1_ref, w2_ref, b2_ref, w3_ref, b3_ref, o_ref):
    def l2n(x):
        nrm = jnp.sqrt(jnp.sum(x * x, axis=1, keepdims=True))
        return x / jnp.maximum(nrm, 1e-12)

    hid = jnp.concatenate([l2n(u1_ref[...]), l2n(u2_ref[...]), l2n(uc_ref[...])], axis=1)
    h = jnp.maximum(jnp.dot(hid, w1_ref[...], preferred_element_type=jnp.float32) + b1_ref[...], 0.0)
    h = jnp.maximum(jnp.dot(h, w2_ref[...], preferred_element_type=jnp.float32) + b2_ref[...], 0.0)
    o_ref[...] = jnp.dot(h, w3_ref[...], preferred_element_type=jnp.float32) + b3_ref[...]


def _mlp(u1, u2, uc, w1, b1, w2, b2, w3, b3, block=512):
    grid = (BATCH // block,)
    return pl.pallas_call(
        _mlp_body,
        grid=grid,
        in_specs=[
            pl.BlockSpec((block, HID), lambda i: (i, 0)),
            pl.BlockSpec((block, HID), lambda i: (i, 0)),
            pl.BlockSpec((block, HID), lambda i: (i, 0)),
            pl.BlockSpec(w1.shape, lambda i: (0, 0)),
            pl.BlockSpec(b1.shape, lambda i: (0,)),
            pl.BlockSpec(w2.shape, lambda i: (0, 0)),
            pl.BlockSpec(b2.shape, lambda i: (0,)),
            pl.BlockSpec(w3.shape, lambda i: (0, 0)),
            pl.BlockSpec(b3.shape, lambda i: (0,)),
        ],
        out_specs=pl.BlockSpec((block, 2), lambda i: (i, 0)),
        out_shape=jax.ShapeDtypeStruct((BATCH, 2), jnp.float32),
    )(u1, u2, uc, w1, b1, w2, b2, w3, b3)


def _seg_sum_cnt(msgs_dsts, n_dst):
    total = None
    cnt = None
    for msg, dst in msgs_dsts:
        s = jax.ops.segment_sum(msg, dst, num_segments=n_dst)
        c = jax.ops.segment_sum(jnp.ones((msg.shape[0],), msg.dtype), dst, num_segments=n_dst)
        total = s if total is None else total + s
        cnt = c if cnt is None else cnt + c
    return total, cnt


def kernel(drug_table, protein_table, cell_table, gnn_w, w1, b1, w2, b2, w3, b3,
           x_drug, x_protein, x_cell, edge_index_dp, edge_index_pd, edge_index_pp,
           edge_index_cp, edge_index_pc, drug1, drug2, cell):
    n_d, n_p, n_c = drug_table.shape[0], protein_table.shape[0], cell_table.shape[0]
    h_d, h_p, h_c = drug_table, protein_table, cell_table

    dp, pd, pp, cp, pc = edge_index_dp, edge_index_pd, edge_index_pp, edge_index_cp, edge_index_pc

    s_d, c_d = _seg_sum_cnt([(h_p[pd[0]], pd[1])], n_d)
    s_p, c_p = _seg_sum_cnt([(h_d[dp[0]], dp[1]), (h_p[pp[0]], pp[1]), (h_c[cp[0]], cp[1])], n_p)
    s_c, c_c = _seg_sum_cnt([(h_p[pc[0]], pc[1])], n_c)
    cd16 = jnp.broadcast_to(c_d[:, None], (n_d, 16))
    cp16 = jnp.broadcast_to(c_p[:, None], (n_p, 16))
    cc16 = jnp.broadcast_to(c_c[:, None], (n_c, 16))
    h_d1 = _layer_update(h_d, s_d, cd16, gnn_w[0])
    h_p1 = _layer_update(h_p, s_p, cp16, gnn_w[0])
    h_c1 = _layer_update(h_c, s_c, cc16, gnn_w[0])

    s_d2, _ = _seg_sum_cnt([(h_p1[pd[0]], pd[1])], n_d)
    s_c2, _ = _seg_sum_cnt([(h_p1[pc[0]], pc[1])], n_c)
    h_d2 = _layer_update(h_d1, s_d2, cd16, gnn_w[1])
    h_c2 = _layer_update(h_c1, s_c2, cc16, gnn_w[1])

    u1 = jnp.take(h_d2, drug1, axis=0)
    u2 = jnp.take(h_d2, drug2, axis=0)
    uc = jnp.take(h_c2, cell, axis=0)
    return _mlp(u1, u2, uc, w1, b1, w2, b2, w3, b3)


# speedup vs baseline: 2.3828x; 2.3828x over previous
import jax
import jax.numpy as jnp
from jax import lax
from jax.experimental import pallas as pl
from jax.experimental.pallas import tpu as pltpu
from jax.experimental.pallas import tpu_sc as plsc

HID = 128
BATCH = 4096
NC = 2
NS = 16
K = 128
EB = 512


def _mesh():
    return plsc.VectorSubcoreMesh(core_axis_name="c", subcore_axis_name="s",
                                  num_cores=NC, num_subcores=NS)


def _params():
    return pltpu.CompilerParams(needs_layout_passes=False)


def _pad_edges(e, epw):
    total = NS * epw
    src = jnp.concatenate([e[0], jnp.zeros((total - e.shape[1],), jnp.int32)])
    dst = jnp.concatenate([e[1], jnp.full((total - e.shape[1],), 2**30, jnp.int32)])
    return src, dst


def _make_sc_seg_sum(table_shapes, rels, program, sum_rows, cnt_rows,
                     acc_rows, acc1d_rows):
    with_counts = bool(cnt_rows)
    n_tab = len(table_shapes)
    n_rel = len(rels)
    trash = acc_rows - 8
    trash1d = acc1d_rows - 8

    def body(*refs):
        tabs = refs[:n_tab]
        e_refs = refs[n_tab:n_tab + 2 * n_rel]
        pos = n_tab + 2 * n_rel
        z128 = refs[pos]; pos += 1
        souts = refs[pos:pos + len(sum_rows)]; pos += len(sum_rows)
        couts = refs[pos:pos + len(cnt_rows)]; pos += len(cnt_rows)
        if with_counts:
            (acc, acc1d, esrc, edst, csrc, cdst, cdstc, rows, ones1, zbuf,
             z1d, cbuf) = refs[pos:]
        else:
            (acc, esrc, edst, csrc, cdst, rows, zbuf) = refs[pos:]
            acc1d = cdstc = ones1 = z1d = cbuf = None

        c = lax.axis_index("c")
        s = lax.axis_index("s")
        wid = s * NC + c
        trash_v = jnp.zeros((16,), jnp.int32) + (trash + lax.rem(wid, 8))
        trash1d_v = jnp.zeros((16,), jnp.int32) + (trash1d + lax.rem(wid, 8))
        zero_v = jnp.zeros((16,), jnp.int32)

        pltpu.sync_copy(z128, zbuf)
        if with_counts:
            def init1(j, _):
                ones1[pl.ds(j * 16, 16)] = jnp.ones((16,), jnp.float32)
                return 0
            lax.fori_loop(0, K // 16, init1, 0)

            def initz(j, _):
                z1d[pl.ds(j * 16, 16)] = jnp.zeros((16,), jnp.float32)
                return 0
            lax.fori_loop(0, 1024 // 16, initz, 0)

        def emit_phase(sum_slot, base, R, rel_ids, cnt_off):
            r16 = R // NS
            lo = base + c * R

            def zero_sum(j, _):
                pltpu.sync_copy(zbuf, acc.at[pl.ds(s * r16 + j * 16, 16)])
                return 0
            lax.fori_loop(0, r16 // 16, zero_sum, 0)
            plsc.subcore_barrier()

            def reset_idx(j, _):
                cdst[pl.ds(j * 16, 16)] = trash_v
                csrc[pl.ds(j * 16, 16)] = zero_v
                if cnt_off is not None:
                    cdstc[pl.ds(j * 16, 16)] = trash1d_v
                return 0
            lax.fori_loop(0, K // 16, reset_idx, 0)

            for rid in rel_ids:
                tslot, epw = rels[rid]
                tab = tabs[tslot]
                src_h = e_refs[2 * rid]
                dst_h = e_refs[2 * rid + 1]

                def flush_op(_tab=tab):
                    pltpu.sync_copy(_tab.at[csrc], rows)
                    pltpu.sync_copy(rows, acc.at[cdst], add=True)
                    if cnt_off is not None:
                        pltpu.sync_copy(ones1, acc1d.at[cdstc], add=True)
                    lax.fori_loop(0, K // 16, reset_idx, 0)

                def stage_body(b, cnt, _flush=flush_op, _src_h=src_h,
                               _dst_h=dst_h, _epw=epw):
                    pltpu.sync_copy(_src_h.at[pl.ds(s * _epw + b * EB, EB)], esrc)
                    pltpu.sync_copy(_dst_h.at[pl.ds(s * _epw + b * EB, EB)], edst)

                    def scan_body(i, cnt):
                        dv = edst[pl.ds(i * 16, 16)]
                        m = (dv >= lo) & (dv < lo + R)
                        mi = jnp.where(m, 1, 0).astype(jnp.int32)
                        p = plsc.cumsum(mi)
                        nm = p[15]

                        def do_f(ct):
                            _flush()
                            return jnp.int32(0)
                        cnt = lax.cond(cnt + nm > K, do_f, lambda ct: ct, cnt)
                        idx = cnt + p - 1
                        plsc.store_scatter(cdst, [idx], dv - lo, mask=m)
                        sv = esrc[pl.ds(i * 16, 16)]
                        plsc.store_scatter(csrc, [idx], sv, mask=m)
                        if cnt_off is not None:
                            plsc.store_scatter(cdstc, [idx], dv - lo + cnt_off, mask=m)
                        return cnt + nm

                    return lax.fori_loop(0, EB // 16, scan_body, cnt)

                cnt_f = lax.fori_loop(0, epw // EB, stage_body, jnp.int32(0))
                lax.cond(cnt_f > 0, flush_op, lambda: None)

            plsc.subcore_barrier()
            obase = base + c * R + s * r16
            pltpu.sync_copy(acc.at[pl.ds(s * r16, r16)],
                            souts[sum_slot].at[pl.ds(obase, r16)])
            plsc.subcore_barrier()

        for op in program:
            if op[0] == "phase":
                _, sum_slot, base, R, rel_ids, cnt_off = op
                emit_phase(sum_slot, base, R, rel_ids, cnt_off)
            elif op[0] == "zero1d":
                _, L = op
                t16 = L // NS
                nfull, rem = t16 // 1024, t16 % 1024
                if nfull:
                    def zero_cnt(j, _, _t16=t16):
                        pltpu.sync_copy(z1d, acc1d.at[pl.ds(s * _t16 + j * 1024, 1024)])
                        return 0
                    lax.fori_loop(0, nfull, zero_cnt, 0)
                if rem:
                    pltpu.sync_copy(z1d.at[pl.ds(0, rem)],
                                    acc1d.at[pl.ds(s * t16 + nfull * 1024, rem)])
            elif op[0] == "wb1d":
                _, cnt_slot, src_off, L, dst_mul = op
                t16 = L // NS
                dbase = dst_mul + c * L + s * t16
                pltpu.sync_copy(acc1d.at[pl.ds(src_off + s * t16, t16)],
                                cbuf.at[pl.ds(0, t16)])
                pltpu.sync_copy(cbuf.at[pl.ds(0, t16)],
                                couts[cnt_slot].at[pl.ds(dbase, t16)])
                plsc.subcore_barrier()

    out_type = [jax.ShapeDtypeStruct((r, HID), jnp.float32) for r in sum_rows]
    out_type += [jax.ShapeDtypeStruct((r,), jnp.float32) for r in cnt_rows]
    scratch = [pltpu.VMEM_SHARED((acc_rows, HID), jnp.float32)]
    if with_counts:
        scratch.append(pltpu.VMEM_SHARED((acc1d_rows,), jnp.float32))
    scratch += [
        pltpu.VMEM((EB,), jnp.int32),
        pltpu.VMEM((EB,), jnp.int32),
        pltpu.VMEM((K,), jnp.int32),
        pltpu.VMEM((K,), jnp.int32),
    ]
    if with_counts:
        scratch.append(pltpu.VMEM((K,), jnp.int32))
    scratch.append(pltpu.VMEM((K, HID), jnp.float32))
    if with_counts:
        scratch.append(pltpu.VMEM((K,), jnp.float32))
    scratch.append(pltpu.VMEM((16, HID), jnp.float32))
    if with_counts:
        scratch.append(pltpu.VMEM((1024,), jnp.float32))
        scratch.append(pltpu.VMEM((1600,), jnp.float32))

    return pl.kernel(body, out_type=out_type, mesh=_mesh(), scratch_types=scratch,
                     compiler_params=_params())


def _sc_batch_gather(h_d2, h_c2, drug1, drug2, cell):
    per = BATCH // (NC * NS)

    def body(hd, hc, i1, i2, ic, o1, o2, oc, idx_v, rows_v):
        c = lax.axis_index("c")
        s = lax.axis_index("s")
        wid = s * NC + c
        base = wid * per
        for (ib, tab, ob) in ((i1, hd, o1), (i2, hd, o2), (ic, hc, oc)):
            pltpu.sync_copy(ib.at[pl.ds(base, per)], idx_v)
            pltpu.sync_copy(tab.at[idx_v], rows_v)
            pltpu.sync_copy(rows_v, ob.at[pl.ds(base, per)])

    out_type = [jax.ShapeDtypeStruct((BATCH, HID), jnp.float32)] * 3
    scratch = [pltpu.VMEM((per,), jnp.int32), pltpu.VMEM((per, HID), jnp.float32)]
    return pl.kernel(body, out_type=out_type, mesh=_mesh(), scratch_types=scratch,
                     compiler_params=_params())(h_d2, h_c2, drug1, drug2, cell)


def _layer_update_body(h_ref, s_ref, cnt_ref, w_ref, o_ref):
    h = h_ref[...]
    sm = s_ref[...]
    deg = jnp.maximum(cnt_ref[...], 1.0)
    x = h + sm / deg
    o_ref[...] = jnp.maximum(jnp.dot(x, w_ref[...], preferred_element_type=jnp.float32), 0.0)


def _layer_update(h, ssum, cnt, W, block=512):
    n = h.shape[0]
    grid = (pl.cdiv(n, block),)
    return pl.pallas_call(
        _layer_update_body,
        grid=grid,
        in_specs=[
            pl.BlockSpec((block, HID), lambda i: (i, 0)),
            pl.BlockSpec((block, HID), lambda i: (i, 0)),
            pl.BlockSpec((block, 1), lambda i: (i, 0)),
            pl.BlockSpec((HID, HID), lambda i: (0, 0)),
        ],
        out_specs=pl.BlockSpec((block, HID), lambda i: (i, 0)),
        out_shape=jax.ShapeDtypeStruct((n, HID), jnp.float32),
    )(h, ssum, cnt, W)


def _mlp_body(u1_ref, u2_ref, uc_ref, w1_ref, b1_ref, w2_ref, b2_ref, w3_ref, b3_ref, o_ref):
    def l2n(x):
        nrm = jnp.sqrt(jnp.sum(x * x, axis=1, keepdims=True))
        return x / jnp.maximum(nrm, 1e-12)

    hid = jnp.concatenate([l2n(u1_ref[...]), l2n(u2_ref[...]), l2n(uc_ref[...])], axis=1)
    h = jnp.maximum(jnp.dot(hid, w1_ref[...], preferred_element_type=jnp.float32) + b1_ref[...], 0.0)
    h = jnp.maximum(jnp.dot(h, w2_ref[...], preferred_element_type=jnp.float32) + b2_ref[...], 0.0)
    o_ref[...] = jnp.dot(h, w3_ref[...], preferred_element_type=jnp.float32) + b3_ref[...]


def _mlp(u1, u2, uc, w1, b1, w2, b2, w3, b3, block=512):
    grid = (BATCH // block,)
    return pl.pallas_call(
        _mlp_body,
        grid=grid,
        in_specs=[
            pl.BlockSpec((block, HID), lambda i: (i, 0)),
            pl.BlockSpec((block, HID), lambda i: (i, 0)),
            pl.BlockSpec((block, HID), lambda i: (i, 0)),
            pl.BlockSpec(w1.shape, lambda i: (0, 0)),
            pl.BlockSpec(b1.shape, lambda i: (0,)),
            pl.BlockSpec(w2.shape, lambda i: (0, 0)),
            pl.BlockSpec(b2.shape, lambda i: (0,)),
            pl.BlockSpec(w3.shape, lambda i: (0, 0)),
            pl.BlockSpec(b3.shape, lambda i: (0,)),
        ],
        out_specs=pl.BlockSpec((block, 2), lambda i: (i, 0)),
        out_shape=jax.ShapeDtypeStruct((BATCH, 2), jnp.float32),
    )(u1, u2, uc, w1, b1, w2, b2, w3, b3)


def kernel(drug_table, protein_table, cell_table, gnn_w, w1, b1, w2, b2, w3, b3,
           x_drug, x_protein, x_cell, edge_index_dp, edge_index_pd, edge_index_pp,
           edge_index_cp, edge_index_pc, drug1, drug2, cell):
    n_d, n_p, n_c = drug_table.shape[0], protein_table.shape[0], cell_table.shape[0]
    h_d, h_p, h_c = drug_table, protein_table, cell_table

    epw_dp = epw_pd = 10240
    epw_pp = 12800
    epw_cp = epw_pc = 3584
    dp_s, dp_d = _pad_edges(edge_index_dp, epw_dp)
    pd_s, pd_d = _pad_edges(edge_index_pd, epw_pd)
    pp_s, pp_d = _pad_edges(edge_index_pp, epw_pp)
    cp_s, cp_d = _pad_edges(edge_index_cp, epw_cp)
    pc_s, pc_d = _pad_edges(edge_index_pc, epw_pc)

    z128 = jnp.zeros((16, HID), jnp.float32)

    seg1 = _make_sc_seg_sum(
        table_shapes=[(n_d, HID), (n_p, HID), (n_c, HID)],
        rels=[(1, epw_pd), (0, epw_dp), (1, epw_pp), (2, epw_cp), (1, epw_pc)],
        program=[
            ("zero1d", 25600),
            ("phase", 1, 0, 12800, [1, 2, 3], 0),
            ("phase", 1, 25600, 12800, [1, 2, 3], 12800),
            ("wb1d", 1, 0, 12800, 0),
            ("wb1d", 1, 12800, 12800, 25600),
            ("zero1d", 5120),
            ("phase", 0, 0, 5120, [0], 0),
            ("wb1d", 0, 0, 5120, 0),
            ("zero1d", 512),
            ("phase", 2, 0, 512, [4], 0),
            ("wb1d", 2, 0, 512, 0),
        ],
        sum_rows=[10240, 51200, 1024],
        cnt_rows=[10240, 51200, 1024],
        acc_rows=12808,
        acc1d_rows=25608,
    )
    s_d, s_p, s_c, c_d, c_p, c_c = seg1(
        h_d, h_p, h_c, pd_s, pd_d, dp_s, dp_d, pp_s, pp_d, cp_s, cp_d, pc_s, pc_d, z128)
    c_d2d, c_p2d, c_c2d = c_d[:, None], c_p[:, None], c_c[:, None]

    h_d1 = _layer_update(h_d, s_d, c_d2d, gnn_w[0])
    h_p1 = _layer_update(h_p, s_p, c_p2d, gnn_w[0])
    h_c1 = _layer_update(h_c, s_c, c_c2d, gnn_w[0])

    seg2 = _make_sc_seg_sum(
        table_shapes=[(n_p, HID)],
        rels=[(0, epw_pd), (0, epw_pc)],
        program=[
            ("phase", 0, 0, 5120, [0], None),
            ("phase", 1, 0, 512, [1], None),
        ],
        sum_rows=[10240, 1024],
        cnt_rows=[],
        acc_rows=5128,
        acc1d_rows=16,
    )
    s_d2, s_c2 = seg2(h_p1, pd_s, pd_d, pc_s, pc_d, z128)

    h_d2 = _layer_update(h_d1, s_d2, c_d2d, gnn_w[1])
    h_c2 = _layer_update(h_c1, s_c2, c_c2d, gnn_w[1])

    u1, u2, uc = _sc_batch_gather(h_d2, h_c2, drug1, drug2, cell)
    return _mlp(u1, u2, uc, w1, b1, w2, b2, w3, b3)
